# baseline (speedup 1.0000x reference)
import jax
import jax.numpy as jnp
from jax.experimental import pallas as pl
from jax.experimental.pallas import tpu as pltpu

_BLOCK_ROWS = 8192


def _copy_body(u_ref, i_ref, ou_ref, oi_ref):
    ou_ref[...] = u_ref[...]
    oi_ref[...] = i_ref[...]


def kernel(user_emb, item_emb):
    n_u, d = user_emb.shape
    n_i, _ = item_emb.shape
    u2 = user_emb.reshape(n_u // 2, 2 * d)
    i2 = item_emb.reshape(n_i // 2, 2 * d)
    grid = (pl.cdiv(u2.shape[0], _BLOCK_ROWS),)
    out_u, out_i = pl.pallas_call(
        _copy_body,
        grid=grid,
        in_specs=[
            pl.BlockSpec((_BLOCK_ROWS, 2 * d), lambda r: (r, 0)),
            pl.BlockSpec((_BLOCK_ROWS, 2 * d), lambda r: (r, 0)),
        ],
        out_specs=[
            pl.BlockSpec((_BLOCK_ROWS, 2 * d), lambda r: (r, 0)),
            pl.BlockSpec((_BLOCK_ROWS, 2 * d), lambda r: (r, 0)),
        ],
        out_shape=[
            jax.ShapeDtypeStruct(u2.shape, user_emb.dtype),
            jax.ShapeDtypeStruct(i2.shape, item_emb.dtype),
        ],
        compiler_params=pltpu.CompilerParams(
            dimension_semantics=("parallel",),
        ),
    )(u2, i2)
    return (out_u.reshape(n_u, d), out_i.reshape(n_i, d))


# speedup vs baseline: 1.2905x; 1.2905x over previous
import jax
import jax.numpy as jnp
from jax import lax
from jax.experimental import pallas as pl
from jax.experimental.pallas import tpu as pltpu
from jax.experimental.pallas import tpu_sc as plsc

_NC = 2
_NS = 16
_NW = _NC * _NS
_CHUNK = 400
_BLOCK_ROWS = 8192


def _sc_body(i_hbm, oi_hbm, bufs, rsem, wsem):
    wid = lax.axis_index("s") * _NC + lax.axis_index("c")

    def chunk_row(t):
        return pl.multiple_of((t * _NW + wid) * _CHUNK, 8)

    nchunks = i_hbm.shape[0] // _CHUNK
    tasks = [chunk_row(t) for t in range(nchunks // _NW)]
    reads, writes = [], []
    for k, off in enumerate(tasks):
        b = k % 2
        reads.append(pltpu.make_async_copy(
            i_hbm.at[pl.ds(off, _CHUNK), :], bufs.at[b], rsem.at[b]))
        writes.append(pltpu.make_async_copy(
            bufs.at[b], oi_hbm.at[pl.ds(off, _CHUNK), :], wsem.at[b]))
    n = len(tasks)
    reads[0].start()
    for k in range(n):
        reads[k].wait()
        if k + 1 < n:
            if k >= 1:
                writes[k - 1].wait()
            reads[k + 1].start()
        writes[k].start()
    writes[n - 1].wait()
    if n >= 2:
        writes[n - 2].wait()

    left = nchunks % _NW
    full = nchunks // _NW
    if left:
        @pl.when(wid < left)
        def _():
            off = pl.multiple_of((full * _NW + wid) * _CHUNK, 8)
            r = pltpu.make_async_copy(
                i_hbm.at[pl.ds(off, _CHUNK), :], bufs.at[0], rsem.at[0])
            w = pltpu.make_async_copy(
                bufs.at[0], oi_hbm.at[pl.ds(off, _CHUNK), :], wsem.at[0])
            r.start()
            r.wait()
            w.start()
            w.wait()


def _tc_body(u_ref, ou_ref):
    ou_ref[...] = u_ref[...]


def kernel(user_emb, item_emb):
    n_u, d = user_emb.shape
    n_i, _ = item_emb.shape

    mesh = plsc.VectorSubcoreMesh(core_axis_name="c", subcore_axis_name="s",
                                  num_cores=_NC, num_subcores=_NS)
    out_i = pl.kernel(
        _sc_body,
        out_type=jax.ShapeDtypeStruct((n_i, d), item_emb.dtype),
        mesh=mesh,
        scratch_types=[
            pltpu.VMEM((2, _CHUNK, 64), jnp.float32),
            pltpu.SemaphoreType.DMA((2,)),
            pltpu.SemaphoreType.DMA((2,)),
        ],
    )(item_emb)

    out_u = pl.pallas_call(
        _tc_body,
        grid=(pl.cdiv(n_u, _BLOCK_ROWS),),
        in_specs=[pl.BlockSpec((_BLOCK_ROWS, d), lambda r: (r, 0))],
        out_specs=pl.BlockSpec((_BLOCK_ROWS, d), lambda r: (r, 0)),
        out_shape=jax.ShapeDtypeStruct((n_u, d), user_emb.dtype),
        compiler_params=pltpu.CompilerParams(
            dimension_semantics=("parallel",),
        ),
    )(user_emb)
    return (out_u, out_i)
